# baseline (speedup 1.0000x reference)
import functools
import math

import jax
import jax.numpy as jnp
from jax import lax
from jax.experimental import pallas as pl
from jax.experimental.pallas import tpu as pltpu
from jax.experimental.pallas import tpu_sc as plsc

_NC = 2
_NS = 16
_NW = _NC * _NS
_LANES = 16


def _pos_encoding(seq_len, d_embed):
    pos = jnp.arange(seq_len, dtype=jnp.float32)
    denom = jnp.exp(
        -jnp.arange(0, d_embed, 2, dtype=jnp.float32) * math.log(10000.0) / d_embed
    )
    phase = pos[:, None] * denom[None, :]
    enc = jnp.zeros((seq_len, d_embed), dtype=jnp.float32)
    enc = enc.at[:, 0::2].set(jnp.sin(phase))
    enc = enc.at[:, 1::2].set(jnp.cos(phase))
    return enc


def _make_sc_embed(B, T, D, idx_cols):
    N = B * T
    rows_w = N // _NW
    seqs_mega = 2
    rows_mega = seqs_mega * T
    megas_w = rows_w // rows_mega
    idx_rows_w = rows_w // idx_cols
    idx_rows_mega = rows_mega // idx_cols
    batches_w = rows_w // T
    scale = float(math.sqrt(D))
    mesh = plsc.VectorSubcoreMesh(core_axis_name="c", subcore_axis_name="s")

    @functools.partial(
        pl.kernel,
        out_type=jax.ShapeDtypeStruct((N // 2, 2 * D), jnp.float32),
        mesh=mesh,
        scratch_types=[
            pltpu.VMEM((idx_rows_w, idx_cols), jnp.int32),
            pltpu.VMEM((rows_w + _LANES,), jnp.int32),
            pltpu.VMEM((seqs_mega, T, 2 * D), jnp.float32),
            pltpu.VMEM((rows_mega // 2, 2 * D), jnp.float32),
            pltpu.VMEM((T, D), jnp.float32),
            pltpu.SemaphoreType.DMA,
        ],
    )
    def k(idx_hbm, off_hbm, table_hbm, pe_hbm, out_hbm,
          idx_v, off_v, raw_v, out_v, pe_v, sem):
        wid = lax.axis_index("s") * _NC + lax.axis_index("c")
        pltpu.sync_copy(pe_hbm, pe_v)
        ibase = pl.multiple_of(wid * idx_rows_w, 8)
        pltpu.sync_copy(idx_hbm.at[pl.ds(ibase, idx_rows_w)], idx_v)
        obase = pl.multiple_of(wid * rows_w, 8)
        pltpu.sync_copy(off_hbm.at[pl.ds(obase, rows_w)],
                        off_v.at[pl.ds(0, rows_w)])

        def mega_body(g, carry):
            cps = [
                pltpu.async_copy(
                    table_hbm.at[idx_v.at[g * idx_rows_mega + j]],
                    raw_v.at[j // seqs_mega,
                             pl.ds((j % seqs_mega) * idx_cols, idx_cols)],
                    sem,
                )
                for j in range(idx_rows_mega)
            ]
            for cp in cps:
                cp.wait()

            def row_body(r, carry2):
                q2 = r // 2
                p2 = r - 2 * q2
                for s in range(seqs_mega):
                    o = off_v[pl.ds(g * rows_mega + s * T + r, _LANES)][0]
                    for j in range(D // _LANES):
                        out_v[s * (T // 2) + q2,
                              pl.ds(p2 * D + j * _LANES, _LANES)] = (
                            raw_v[s, r, pl.ds(o + j * _LANES, _LANES)] * scale
                            + pe_v[r, pl.ds(j * _LANES, _LANES)]
                        )
                return carry2

            lax.fori_loop(0, T, row_body, 0)
            base2 = pl.multiple_of((wid * rows_w + g * rows_mega) // 2, 8)
            pltpu.sync_copy(out_v, out_hbm.at[pl.ds(base2, rows_mega // 2)])
            return carry

        lax.fori_loop(0, megas_w, mega_body, 0)

    return k


def kernel(indices, embed_weight):
    B, T = indices.shape
    V, D = embed_weight.shape
    N = B * T
    idx_cols = 100
    pe = _pos_encoding(T, D)
    table128 = embed_weight.reshape(V // 2, 2 * D)
    flat = indices.reshape(-1)
    idx_half = (flat // 2).reshape(N // idx_cols, idx_cols)
    offs = ((flat % 2) * D).astype(jnp.int32)
    out2 = _make_sc_embed(B, T, D, idx_cols)(idx_half, offs, table128, pe)
    return out2.reshape(B, T, D)


# speedup vs baseline: 1.1804x; 1.1804x over previous
import functools
import math

import jax
import jax.numpy as jnp
from jax import lax
from jax.experimental import pallas as pl
from jax.experimental.pallas import tpu as pltpu
from jax.experimental.pallas import tpu_sc as plsc

_NC = 2
_NS = 16
_NW = _NC * _NS
_LANES = 16


def _pos_encoding(seq_len, d_embed):
    pos = jnp.arange(seq_len, dtype=jnp.float32)
    denom = jnp.exp(
        -jnp.arange(0, d_embed, 2, dtype=jnp.float32) * math.log(10000.0) / d_embed
    )
    phase = pos[:, None] * denom[None, :]
    enc = jnp.zeros((seq_len, d_embed), dtype=jnp.float32)
    enc = enc.at[:, 0::2].set(jnp.sin(phase))
    enc = enc.at[:, 1::2].set(jnp.cos(phase))
    return enc


def _make_sc_embed(B, T, D, idx_cols):
    N = B * T
    W = 2 * D
    rows_w = N // _NW
    seqs_mega = 2
    rows_mega = seqs_mega * T
    megas_w = rows_w // rows_mega
    idx_rows_w = rows_w // idx_cols
    idx_rows_mega = rows_mega // idx_cols
    scale = float(math.sqrt(D))
    mesh = plsc.VectorSubcoreMesh(core_axis_name="c", subcore_axis_name="s")

    @functools.partial(
        pl.kernel,
        out_type=jax.ShapeDtypeStruct((N // 2, W), jnp.float32),
        mesh=mesh,
        scratch_types=[
            pltpu.VMEM((idx_rows_w, idx_cols), jnp.int32),
            pltpu.VMEM((seqs_mega, T, W), jnp.float32),
            pltpu.VMEM((rows_mega // 2, W), jnp.float32),
            pltpu.VMEM((T, D), jnp.float32),
            pltpu.SemaphoreType.DMA,
        ],
    )
    def k(idx_hbm, table_hbm, pe_hbm, out_hbm,
          idx_v, raw_v, out_v, pe_v, sem):
        wid = lax.axis_index("s") * _NC + lax.axis_index("c")
        pltpu.sync_copy(pe_hbm, pe_v)
        ibase = pl.multiple_of(wid * idx_rows_w, 8)
        pltpu.sync_copy(idx_hbm.at[pl.ds(ibase, idx_rows_w)], idx_v)

        def mega_body(g, carry):
            cps = [
                pltpu.async_copy(
                    table_hbm.at[idx_v.at[g * idx_rows_mega + j]],
                    raw_v.at[j // seqs_mega,
                             pl.ds((j % seqs_mega) * idx_cols, idx_cols)],
                    sem,
                )
                for j in range(idx_rows_mega)
            ]
            for cp in cps:
                cp.wait()

            def row_body(r2, carry2):
                for par in range(2):
                    r = 2 * r2 + par
                    for j in range(D // _LANES):
                        pe_j = pe_v[r, pl.ds(j * _LANES, _LANES)]
                        for s in range(seqs_mega):
                            out_v[s * (T // 2) + r2,
                                  pl.ds(par * D + j * _LANES, _LANES)] = (
                                raw_v[s, r, pl.ds(j * _LANES, _LANES)] * scale
                                + pe_j
                            )
                return carry2

            lax.fori_loop(0, T // 2, row_body, 0)
            base2 = pl.multiple_of((wid * rows_w + g * rows_mega) // 2, 8)
            pltpu.sync_copy(out_v, out_hbm.at[pl.ds(base2, rows_mega // 2)])
            return carry

        lax.fori_loop(0, megas_w, mega_body, 0)

    return k


def kernel(indices, embed_weight):
    B, T = indices.shape
    V, D = embed_weight.shape
    N = B * T
    idx_cols = 100
    pe = _pos_encoding(T, D)
    table_pad = jnp.pad(embed_weight, ((0, 0), (0, D)))
    idx2d = indices.reshape(N // idx_cols, idx_cols)
    out2 = _make_sc_embed(B, T, D, idx_cols)(idx2d, table_pad, pe)
    return out2.reshape(B, T, D)


# speedup vs baseline: 1.3229x; 1.1208x over previous
import functools
import math

import jax
import jax.numpy as jnp
from jax import lax
from jax.experimental import pallas as pl
from jax.experimental.pallas import tpu as pltpu
from jax.experimental.pallas import tpu_sc as plsc

_NC = 2
_NS = 16
_NW = _NC * _NS
_LANES = 16


def _pos_encoding(seq_len, d_embed):
    pos = jnp.arange(seq_len, dtype=jnp.float32)
    denom = jnp.exp(
        -jnp.arange(0, d_embed, 2, dtype=jnp.float32) * math.log(10000.0) / d_embed
    )
    phase = pos[:, None] * denom[None, :]
    enc = jnp.zeros((seq_len, d_embed), dtype=jnp.float32)
    enc = enc.at[:, 0::2].set(jnp.sin(phase))
    enc = enc.at[:, 1::2].set(jnp.cos(phase))
    return enc


def _make_sc_embed(B, T, D, idx_cols):
    N = B * T
    rows_w = N // _NW
    seqs_mega = 8
    rows_mega = seqs_mega * T
    megas_w = rows_w // rows_mega
    idx_rows_w = rows_w // idx_cols
    idx_rows_mega = rows_mega // idx_cols
    batches_w = rows_w // T
    scale = float(math.sqrt(D))
    mesh = plsc.VectorSubcoreMesh(core_axis_name="c", subcore_axis_name="s")

    @functools.partial(
        pl.kernel,
        out_type=jax.ShapeDtypeStruct((B, T, D), jnp.float32),
        mesh=mesh,
        compiler_params=pltpu.CompilerParams(use_tc_tiling_on_sc=False),
        scratch_types=[
            pltpu.VMEM((idx_rows_w, idx_cols), jnp.int32),
            pltpu.VMEM((seqs_mega, T, D), jnp.float32),
            pltpu.VMEM((T, D), jnp.float32),
            pltpu.SemaphoreType.DMA,
        ],
    )
    def k(idx_hbm, table_hbm, pe_hbm, out_hbm, idx_v, rows_v, pe_v, sem):
        wid = lax.axis_index("s") * _NC + lax.axis_index("c")
        pltpu.sync_copy(pe_hbm, pe_v)
        pltpu.sync_copy(idx_hbm.at[pl.ds(wid * idx_rows_w, idx_rows_w)], idx_v)

        def mega_body(g, carry):
            cps = [
                pltpu.async_copy(
                    table_hbm.at[idx_v.at[g * idx_rows_mega + j]],
                    rows_v.at[j // (idx_rows_mega // seqs_mega),
                              pl.ds((j % (idx_rows_mega // seqs_mega))
                                    * idx_cols, idx_cols)],
                    sem,
                )
                for j in range(idx_rows_mega)
            ]
            for cp in cps:
                cp.wait()

            def row_body(r, carry2):
                for j in range(D // _LANES):
                    sl = pl.ds(j * _LANES, _LANES)
                    pe_j = pe_v[r, sl]
                    for s in range(seqs_mega):
                        rows_v[s, r, sl] = rows_v[s, r, sl] * scale + pe_j
                return carry2

            lax.fori_loop(0, T, row_body, 0)
            b0 = wid * batches_w + g * seqs_mega
            pltpu.sync_copy(rows_v, out_hbm.at[pl.ds(b0, seqs_mega)])
            return carry

        lax.fori_loop(0, megas_w, mega_body, 0)

    return k


def kernel(indices, embed_weight):
    B, T = indices.shape
    V, D = embed_weight.shape
    N = B * T
    idx_cols = 100
    pe = _pos_encoding(T, D)
    idx2d = indices.reshape(N // idx_cols, idx_cols)
    return _make_sc_embed(B, T, D, idx_cols)(idx2d, embed_weight, pe)
